# baseline (speedup 1.0000x reference)
import functools

import jax
import jax.numpy as jnp
from jax import lax
from jax.experimental import pallas as pl
from jax.experimental.pallas import tpu as pltpu
from jax.experimental.pallas import tpu_sc as plsc

_N_BINS = 10
_NC = 2
_NS = 16
_NW = _NC * _NS
_CH = 32
_L = 16


def _sc_body(probas_hbm, labels_hbm, conf_hbm, corr_hbm,
             buf0, buf1, lab_v, conf_v, corr_v, sems):
    n, c = probas_hbm.shape
    rows_per_w = n // _NW
    nchunks = rows_per_w // _CH
    wid = lax.axis_index("s") * _NC + lax.axis_index("c")
    base_row = wid * rows_per_w

    pltpu.make_async_copy(
        labels_hbm.at[pl.ds(base_row, rows_per_w)], lab_v, sems.at[0]
    ).start()
    pltpu.make_async_copy(
        labels_hbm.at[pl.ds(base_row, rows_per_w)], lab_v, sems.at[0]
    ).wait()

    bufs = (buf0, buf1)

    def start_chunk(i, slot):
        pltpu.make_async_copy(
            probas_hbm.at[pl.ds(base_row + i * _CH, _CH), :],
            bufs[slot], sems.at[slot]
        ).start()

    def wait_chunk(i, slot):
        pltpu.make_async_copy(
            probas_hbm.at[pl.ds(base_row + i * _CH, _CH), :],
            bufs[slot], sems.at[slot]
        ).wait()

    start_chunk(jnp.int32(0), 0)
    start_chunk(jnp.int32(1), 1)

    lanes = lax.iota(jnp.int32, _L)

    def process_chunk(chunk_idx, slot):
        buf = bufs[slot]
        for grp in range(_CH // _L):
            idx_r = jnp.full((_L,), grp * _L, jnp.int32) + lanes
            ms = [None] * 4
            idxs = [None] * 4
            for k in range(4):
                idxs[k] = jnp.full((_L,), k, jnp.int32)
                ms[k] = plsc.load_gather(buf, [idx_r, idxs[k]])

            def col_body(t, carry):
                i0, i1, i2, i3, m0, m1, m2, m3 = carry
                i0 = i0 + 4
                i1 = i1 + 4
                i2 = i2 + 4
                i3 = i3 + 4
                m0 = jnp.maximum(m0, plsc.load_gather(buf, [idx_r, i0]))
                m1 = jnp.maximum(m1, plsc.load_gather(buf, [idx_r, i1]))
                m2 = jnp.maximum(m2, plsc.load_gather(buf, [idx_r, i2]))
                m3 = jnp.maximum(m3, plsc.load_gather(buf, [idx_r, i3]))
                return (i0, i1, i2, i3, m0, m1, m2, m3)

            carry = (idxs[0], idxs[1], idxs[2], idxs[3],
                     ms[0], ms[1], ms[2], ms[3])
            carry = lax.fori_loop(1, c // 4, col_body, carry, unroll=4)
            m = jnp.maximum(jnp.maximum(carry[4], carry[5]),
                            jnp.maximum(carry[6], carry[7]))

            local0 = chunk_idx * _CH + grp * _L
            lab16 = lab_v[pl.ds(local0, _L)]
            g = plsc.load_gather(buf, [idx_r, lab16])
            cf = jnp.where(g == m, jnp.float32(1.0), jnp.float32(0.0))
            conf_v[pl.ds(local0, _L)] = m
            corr_v[pl.ds(local0, _L)] = cf

    def pair_body(p, _):
        for k in range(2):
            i = 2 * p + k
            wait_chunk(i, k)
            process_chunk(i, k)

            @pl.when(i + 2 < nchunks)
            def _():
                start_chunk(i + 2, k)
        return _

    lax.fori_loop(0, nchunks // 2, pair_body, None)

    pltpu.make_async_copy(
        conf_v, conf_hbm.at[pl.ds(base_row, rows_per_w)], sems.at[0]
    ).start()
    pltpu.make_async_copy(
        conf_v, conf_hbm.at[pl.ds(base_row, rows_per_w)], sems.at[0]
    ).wait()
    pltpu.make_async_copy(
        corr_v, corr_hbm.at[pl.ds(base_row, rows_per_w)], sems.at[0]
    ).start()
    pltpu.make_async_copy(
        corr_v, corr_hbm.at[pl.ds(base_row, rows_per_w)], sems.at[0]
    ).wait()


def _sc_stage(probas, labels):
    n, c = probas.shape
    rows_per_w = n // _NW
    mesh = plsc.VectorSubcoreMesh(core_axis_name="c", subcore_axis_name="s")
    fn = pl.kernel(
        _sc_body,
        out_type=[
            jax.ShapeDtypeStruct((n,), jnp.float32),
            jax.ShapeDtypeStruct((n,), jnp.float32),
        ],
        mesh=mesh,
        scratch_types=[
            pltpu.VMEM((_CH, c), jnp.float32),
            pltpu.VMEM((_CH, c), jnp.float32),
            pltpu.VMEM((rows_per_w,), jnp.int32),
            pltpu.VMEM((rows_per_w,), jnp.float32),
            pltpu.VMEM((rows_per_w,), jnp.float32),
            pltpu.SemaphoreType.DMA((2,)),
        ],
        compiler_params=pltpu.CompilerParams(needs_layout_passes=False),
    )
    return fn(probas, labels)


def _ce_finish_kernel(bins_ref, conf_ref, corr_ref, out_ref):
    conf = conf_ref[...]
    corr = corr_ref[...]
    bins = [bins_ref[b] for b in range(_N_BINS + 1)]
    total = jnp.float32(0.0)
    cnts = []
    cfs = []
    acs = []
    for b in range(_N_BINS):
        mb = ((conf > bins[b]) & (conf <= bins[b + 1])).astype(jnp.float32)
        cnt_b = jnp.sum(mb)
        cnts.append(cnt_b)
        cfs.append(jnp.sum(mb * conf))
        acs.append(jnp.sum(mb * corr))
        total = total + cnt_b
    ce2 = jnp.float32(0.0)
    for b in range(_N_BINS):
        denom = jnp.maximum(cnts[b], 1.0)
        diff = cfs[b] / denom - acs[b] / denom
        term = (cnts[b] / total) * diff * diff
        ce2 = ce2 + jnp.where(cnts[b] > 0, term, 0.0)
    out_ref[...] = jnp.sqrt(jnp.broadcast_to(ce2, (1, 1)))


def _finish(conf, corr):
    s = conf.shape[0]
    conf2 = conf.reshape(s // 128, 128)
    corr2 = corr.reshape(s // 128, 128)
    bins = jnp.linspace(0.0, 1.0, _N_BINS + 1)
    out = pl.pallas_call(
        _ce_finish_kernel,
        in_specs=[
            pl.BlockSpec(memory_space=pltpu.MemorySpace.SMEM),
            pl.BlockSpec(memory_space=pltpu.MemorySpace.VMEM),
            pl.BlockSpec(memory_space=pltpu.MemorySpace.VMEM),
        ],
        out_shape=jax.ShapeDtypeStruct((1, 1), jnp.float32),
    )(bins, conf2, corr2)
    return out[0, 0]


def kernel(probas, labels):
    conf, corr = _sc_stage(probas, labels)
    return _finish(conf, corr)


# speedup vs baseline: 1.0837x; 1.0837x over previous
import functools

import jax
import jax.numpy as jnp
from jax import lax
from jax.experimental import pallas as pl
from jax.experimental.pallas import tpu as pltpu
from jax.experimental.pallas import tpu_sc as plsc

_N_BINS = 10
_NC = 2
_NS = 16
_NW = _NC * _NS
_CH = 32
_L = 16


def _sc_body(probas_hbm, labels_hbm, conf_hbm, corr_hbm,
             buf0, buf1, lab_v, conf_v, corr_v, sems):
    n, c = probas_hbm.shape
    rows_per_w = n // _NW
    nchunks = rows_per_w // _CH
    wid = lax.axis_index("s") * _NC + lax.axis_index("c")
    base_row = wid * rows_per_w

    pltpu.make_async_copy(
        labels_hbm.at[pl.ds(base_row, rows_per_w)], lab_v, sems.at[0]
    ).start()
    pltpu.make_async_copy(
        labels_hbm.at[pl.ds(base_row, rows_per_w)], lab_v, sems.at[0]
    ).wait()

    bufs = (buf0, buf1)

    def start_chunk(i, slot):
        pltpu.make_async_copy(
            probas_hbm.at[pl.ds(base_row + i * _CH, _CH), :],
            bufs[slot], sems.at[slot]
        ).start()

    def wait_chunk(i, slot):
        pltpu.make_async_copy(
            probas_hbm.at[pl.ds(base_row + i * _CH, _CH), :],
            bufs[slot], sems.at[slot]
        ).wait()

    start_chunk(jnp.int32(0), 0)
    start_chunk(jnp.int32(1), 1)

    lanes = lax.iota(jnp.int32, _L)

    def process_chunk(chunk_idx, slot):
        buf = bufs[slot]
        for grp in range(_CH // _L):
            idx_r = jnp.full((_L,), grp * _L, jnp.int32) + lanes
            neg = jnp.full((_L,), -jnp.inf, jnp.float32)

            @plsc.parallel_loop(0, c, step=4, unroll=8,
                                carry=(neg, neg, neg, neg))
            def col_loop(j, ms):
                m0, m1, m2, m3 = ms
                z = jnp.full((_L,), 0, jnp.int32) + j
                m0 = jnp.maximum(m0, plsc.load_gather(buf, [idx_r, z]))
                m1 = jnp.maximum(m1, plsc.load_gather(buf, [idx_r, z + 1]))
                m2 = jnp.maximum(m2, plsc.load_gather(buf, [idx_r, z + 2]))
                m3 = jnp.maximum(m3, plsc.load_gather(buf, [idx_r, z + 3]))
                return (m0, m1, m2, m3)

            m0, m1, m2, m3 = col_loop
            m = jnp.maximum(jnp.maximum(m0, m1), jnp.maximum(m2, m3))

            local0 = chunk_idx * _CH + grp * _L
            lab16 = lab_v[pl.ds(local0, _L)]
            g = plsc.load_gather(buf, [idx_r, lab16])
            cf = jnp.where(g == m, jnp.float32(1.0), jnp.float32(0.0))
            conf_v[pl.ds(local0, _L)] = m
            corr_v[pl.ds(local0, _L)] = cf

    def pair_body(p, _):
        for k in range(2):
            i = 2 * p + k
            wait_chunk(i, k)
            process_chunk(i, k)

            @pl.when(i + 2 < nchunks)
            def _():
                start_chunk(i + 2, k)
        return _

    lax.fori_loop(0, nchunks // 2, pair_body, None)

    pltpu.make_async_copy(
        conf_v, conf_hbm.at[pl.ds(base_row, rows_per_w)], sems.at[0]
    ).start()
    pltpu.make_async_copy(
        conf_v, conf_hbm.at[pl.ds(base_row, rows_per_w)], sems.at[0]
    ).wait()
    pltpu.make_async_copy(
        corr_v, corr_hbm.at[pl.ds(base_row, rows_per_w)], sems.at[0]
    ).start()
    pltpu.make_async_copy(
        corr_v, corr_hbm.at[pl.ds(base_row, rows_per_w)], sems.at[0]
    ).wait()


def _sc_stage(probas, labels):
    n, c = probas.shape
    rows_per_w = n // _NW
    mesh = plsc.VectorSubcoreMesh(core_axis_name="c", subcore_axis_name="s")
    fn = pl.kernel(
        _sc_body,
        out_type=[
            jax.ShapeDtypeStruct((n,), jnp.float32),
            jax.ShapeDtypeStruct((n,), jnp.float32),
        ],
        mesh=mesh,
        scratch_types=[
            pltpu.VMEM((_CH, c), jnp.float32),
            pltpu.VMEM((_CH, c), jnp.float32),
            pltpu.VMEM((rows_per_w,), jnp.int32),
            pltpu.VMEM((rows_per_w,), jnp.float32),
            pltpu.VMEM((rows_per_w,), jnp.float32),
            pltpu.SemaphoreType.DMA((2,)),
        ],
        compiler_params=pltpu.CompilerParams(needs_layout_passes=False),
    )
    return fn(probas, labels)


def _ce_finish_kernel(bins_ref, conf_ref, corr_ref, out_ref):
    conf = conf_ref[...]
    corr = corr_ref[...]
    bins = [bins_ref[b] for b in range(_N_BINS + 1)]
    total = jnp.float32(0.0)
    cnts = []
    cfs = []
    acs = []
    for b in range(_N_BINS):
        mb = ((conf > bins[b]) & (conf <= bins[b + 1])).astype(jnp.float32)
        cnt_b = jnp.sum(mb)
        cnts.append(cnt_b)
        cfs.append(jnp.sum(mb * conf))
        acs.append(jnp.sum(mb * corr))
        total = total + cnt_b
    ce2 = jnp.float32(0.0)
    for b in range(_N_BINS):
        denom = jnp.maximum(cnts[b], 1.0)
        diff = cfs[b] / denom - acs[b] / denom
        term = (cnts[b] / total) * diff * diff
        ce2 = ce2 + jnp.where(cnts[b] > 0, term, 0.0)
    out_ref[...] = jnp.sqrt(jnp.broadcast_to(ce2, (1, 1)))


def _finish(conf, corr):
    s = conf.shape[0]
    conf2 = conf.reshape(s // 128, 128)
    corr2 = corr.reshape(s // 128, 128)
    bins = jnp.linspace(0.0, 1.0, _N_BINS + 1)
    out = pl.pallas_call(
        _ce_finish_kernel,
        in_specs=[
            pl.BlockSpec(memory_space=pltpu.MemorySpace.SMEM),
            pl.BlockSpec(memory_space=pltpu.MemorySpace.VMEM),
            pl.BlockSpec(memory_space=pltpu.MemorySpace.VMEM),
        ],
        out_shape=jax.ShapeDtypeStruct((1, 1), jnp.float32),
    )(bins, conf2, corr2)
    return out[0, 0]


def kernel(probas, labels):
    conf, corr = _sc_stage(probas, labels)
    return _finish(conf, corr)


# speedup vs baseline: 1.8292x; 1.6879x over previous
import jax
import jax.numpy as jnp
from jax.experimental import pallas as pl
from jax.experimental.pallas import tpu as pltpu

_N_BINS = 10
_NSLICE = 8
_BLOCK_ROWS = 1024


def _slice_kernel(probas_ref, labels_ref, lo_ref, hi_ref, out_ref,
                  cnt_ref, conf_ref, acc_ref):
    i = pl.program_id(0)
    nsteps = pl.num_programs(0)

    @pl.when(i == 0)
    def _init():
        cnt_ref[...] = jnp.zeros_like(cnt_ref)
        conf_ref[...] = jnp.zeros_like(conf_ref)
        acc_ref[...] = jnp.zeros_like(acc_ref)

    r = _BLOCK_ROWS
    c = probas_ref.shape[1]
    x = probas_ref[pl.ds(i * r, r), :]
    m = jnp.max(x, axis=-1, keepdims=True)
    iota = jax.lax.broadcasted_iota(jnp.int32, (r, c), 1)
    idx = jnp.min(jnp.where(x == m, iota, c), axis=-1, keepdims=True)
    correct = (idx == labels_ref[pl.ds(i * r, r), :]).astype(jnp.float32)

    lo = lo_ref[...]
    hi = hi_ref[...]
    in_bin = ((m > lo) & (m <= hi)).astype(jnp.float32)
    cnt_ref[...] += jnp.sum(in_bin, axis=0, keepdims=True)
    conf_ref[...] += jnp.sum(in_bin * m, axis=0, keepdims=True)
    acc_ref[...] += jnp.sum(in_bin * correct, axis=0, keepdims=True)

    @pl.when(i == nsteps - 1)
    def _finish():
        out_ref[...] = jnp.concatenate(
            [cnt_ref[...], conf_ref[...], acc_ref[...]], axis=0)


def _run_slice(probas_s, labels_s, lo, hi):
    s, c = probas_s.shape
    return pl.pallas_call(
        _slice_kernel,
        grid=(s // _BLOCK_ROWS,),
        in_specs=[
            pl.BlockSpec(memory_space=pltpu.MemorySpace.VMEM),
            pl.BlockSpec(memory_space=pltpu.MemorySpace.VMEM),
            pl.BlockSpec(memory_space=pltpu.MemorySpace.VMEM),
            pl.BlockSpec(memory_space=pltpu.MemorySpace.VMEM),
        ],
        out_specs=pl.BlockSpec(memory_space=pltpu.MemorySpace.VMEM),
        out_shape=jax.ShapeDtypeStruct((3, _N_BINS), jnp.float32),
        scratch_shapes=[
            pltpu.VMEM((1, _N_BINS), jnp.float32),
            pltpu.VMEM((1, _N_BINS), jnp.float32),
            pltpu.VMEM((1, _N_BINS), jnp.float32),
        ],
        compiler_params=pltpu.CompilerParams(
            dimension_semantics=("arbitrary",),
            vmem_limit_bytes=100 * 1024 * 1024,
        ),
    )(probas_s, labels_s, lo, hi)


def _ce_finish_kernel(psum_ref, out_ref):
    psum = psum_ref[...]
    sums = jnp.sum(psum, axis=0)
    cnt = sums[0:1, :]
    conf = sums[1:2, :]
    acc = sums[2:3, :]
    total = jnp.sum(cnt)
    valid = (cnt > 0).astype(jnp.float32)
    denom = jnp.maximum(cnt, 1.0)
    confs = conf / denom
    accs = acc / denom
    terms = (cnt / total) * (confs - accs) ** 2 * valid
    out_ref[...] = jnp.sum(terms, axis=1, keepdims=True) ** 0.5


def kernel(probas, labels):
    n, c = probas.shape
    s = n // _NSLICE
    bins = jnp.linspace(0.0, 1.0, _N_BINS + 1)
    lo = bins[:-1].reshape(1, _N_BINS)
    hi = bins[1:].reshape(1, _N_BINS)
    labels2d = labels.reshape(n, 1)

    psums = [
        _run_slice(
            jax.lax.slice_in_dim(probas, q * s, (q + 1) * s, axis=0),
            jax.lax.slice_in_dim(labels2d, q * s, (q + 1) * s, axis=0),
            lo, hi)
        for q in range(_NSLICE)
    ]
    psum = jnp.stack(psums)

    out = pl.pallas_call(
        _ce_finish_kernel,
        out_shape=jax.ShapeDtypeStruct((1, 1), jnp.float32),
    )(psum)
    return out[0, 0]


# speedup vs baseline: 12.6269x; 6.9028x over previous
import jax
import jax.numpy as jnp
from jax.experimental import pallas as pl
from jax.experimental.pallas import tpu as pltpu

_N_BINS = 10
_W = 2048


def _ce_kernel(xt_ref, labels_ref, lo_ref, hi_ref, out_ref,
               cnt_ref, conf_ref, acc_ref):
    i = pl.program_id(0)
    nsteps = pl.num_programs(0)

    @pl.when(i == 0)
    def _init():
        cnt_ref[...] = jnp.zeros_like(cnt_ref)
        conf_ref[...] = jnp.zeros_like(conf_ref)
        acc_ref[...] = jnp.zeros_like(acc_ref)

    x = xt_ref[...]
    c, w = x.shape
    m = jnp.max(x, axis=0, keepdims=True)
    iota = jax.lax.broadcasted_iota(jnp.int32, (c, w), 0).astype(jnp.float32)
    idx = jnp.min(jnp.where(x == m, iota, jnp.float32(c)),
                  axis=0, keepdims=True)
    lab = labels_ref[0].astype(jnp.float32)
    correct = (idx == lab).astype(jnp.float32)

    lo = lo_ref[...]
    hi = hi_ref[...]
    mask = ((m > lo) & (m <= hi)).astype(jnp.float32)
    cnt_ref[...] += mask
    conf_ref[...] += mask * m
    acc_ref[...] += mask * correct

    @pl.when(i == nsteps - 1)
    def _finish():
        cnt = jnp.sum(cnt_ref[...], axis=1, keepdims=True)
        conf = jnp.sum(conf_ref[...], axis=1, keepdims=True)
        acc = jnp.sum(acc_ref[...], axis=1, keepdims=True)
        total = jnp.sum(cnt)
        valid = (cnt > 0).astype(jnp.float32)
        denom = jnp.maximum(cnt, 1.0)
        terms = (cnt / total) * (conf / denom - acc / denom) ** 2 * valid
        out_ref[...] = jnp.sqrt(
            jnp.broadcast_to(jnp.sum(terms), (1, 1)))


def kernel(probas, labels):
    n, c = probas.shape
    xt = probas.T
    nb = n // _W
    labels3 = labels.reshape(nb, 1, _W)
    bins = jnp.linspace(0.0, 1.0, _N_BINS + 1)
    pad = jnp.full((16 - _N_BINS,), 2.0, jnp.float32)
    lo = jnp.concatenate([bins[:-1], pad]).reshape(16, 1)
    hi = jnp.concatenate([bins[1:], pad]).reshape(16, 1)

    out = pl.pallas_call(
        _ce_kernel,
        grid=(nb,),
        in_specs=[
            pl.BlockSpec((c, _W), lambda i: (0, i)),
            pl.BlockSpec((1, 1, _W), lambda i: (i, 0, 0)),
            pl.BlockSpec((16, 1), lambda i: (0, 0)),
            pl.BlockSpec((16, 1), lambda i: (0, 0)),
        ],
        out_specs=pl.BlockSpec((1, 1), lambda i: (0, 0)),
        out_shape=jax.ShapeDtypeStruct((1, 1), jnp.float32),
        scratch_shapes=[
            pltpu.VMEM((16, _W), jnp.float32),
            pltpu.VMEM((16, _W), jnp.float32),
            pltpu.VMEM((16, _W), jnp.float32),
        ],
        compiler_params=pltpu.CompilerParams(
            dimension_semantics=("arbitrary",),
        ),
    )(xt, labels3, lo, hi)
    return out[0, 0]
